# baseline (speedup 1.0000x reference)
import functools

import jax
import jax.numpy as jnp
from jax import lax
from jax.experimental import pallas as pl
from jax.experimental.pallas import tpu as pltpu
from jax.experimental.pallas import tpu_sc as plsc

B, S, D = 4, 8192, 1024
B_SC = 3
NC, NS = 2, 16
NW = NC * NS
S_PER_W = S // NW
CHUNK = 8
N_CHUNKS = S_PER_W // CHUNK
LANES = 16
S_BLK = 512


def _sc_body(x_hbm, pe_hbm, out_hbm,
             peb0, peb1,
             xb00, xb01, xb02,
             xb10, xb11, xb12,
             pe_sem0, pe_sem1, in_sem0, in_sem1, out_sem):
    peb = (peb0, peb1)
    xb = ((xb00, xb01, xb02), (xb10, xb11, xb12))
    pe_sem = (pe_sem0, pe_sem1)
    in_sem = (in_sem0, in_sem1)

    wid = lax.axis_index("s") * NC + lax.axis_index("c")
    base = wid * S_PER_W

    def pe_row(c):
        return pl.multiple_of(base + c * CHUNK, 8)

    def x_row(c, b):
        return pl.multiple_of(b * S + base + c * CHUNK, 8)

    def issue_pe(c, p):
        pltpu.async_copy(pe_hbm.at[pl.ds(pe_row(c), CHUNK)], peb[p], pe_sem[p])

    def issue_in(c, p):
        for b in range(B_SC):
            pltpu.async_copy(x_hbm.at[pl.ds(x_row(c, b), CHUNK)], xb[p][b],
                             in_sem[p])

    def wait_pe(p):
        pltpu.make_async_copy(pe_hbm.at[pl.ds(0, CHUNK)], peb[p],
                              pe_sem[p]).wait()

    def wait_in(p):
        for b in range(B_SC):
            pltpu.make_async_copy(x_hbm.at[pl.ds(0, CHUNK)], xb[p][b],
                                  in_sem[p]).wait()

    def drain_outs():
        for b in range(B_SC):
            pltpu.make_async_copy(x_hbm.at[pl.ds(0, CHUNK)], xb[0][b],
                                  out_sem).wait()

    def chunk_step(c, p):
        wait_pe(p)

        @pl.when(c + 1 < N_CHUNKS)
        def _():
            issue_pe(c + 1, 1 - p)

        wait_in(p)

        @pl.when(c > 0)
        def _():
            drain_outs()

        @pl.when(c + 1 < N_CHUNKS)
        def _():
            issue_in(c + 1, 1 - p)

        bufs = xb[p]
        pbuf = peb[p]

        for r in range(CHUNK):
            @plsc.parallel_loop(0, D, LANES, unroll=4)
            def _(i, r=r):
                j = pl.multiple_of(i, 8)
                pe_slice = pbuf[r, pl.ds(j, LANES)]
                for b in range(B_SC):
                    bufs[b][r, pl.ds(j, LANES)] = (
                        bufs[b][r, pl.ds(j, LANES)] + pe_slice)

        for b in range(B_SC):
            pltpu.async_copy(bufs[b], out_hbm.at[pl.ds(x_row(c, b), CHUNK)],
                             out_sem)

    issue_pe(0, 0)
    issue_in(0, 0)

    def loop_body(t, carry):
        chunk_step(2 * t, 0)
        chunk_step(2 * t + 1, 1)
        return carry

    lax.fori_loop(0, N_CHUNKS // 2, loop_body, 0)
    drain_outs()


def _tc_body(x_ref, pe_ref, o_ref):
    o_ref[...] = x_ref[...] + pe_ref[...]


@jax.jit
def kernel(x, pe_table):
    x2d = x.reshape(B * S, D)

    mesh = plsc.VectorSubcoreMesh(core_axis_name="c", subcore_axis_name="s")
    sc_k = functools.partial(
        pl.kernel,
        mesh=mesh,
        out_type=jax.ShapeDtypeStruct((B_SC * S, D), jnp.float32),
        scratch_types=(
            [pltpu.VMEM((CHUNK, D), jnp.float32)] * 2
            + [pltpu.VMEM((CHUNK, D), jnp.float32)] * 6
            + [pltpu.SemaphoreType.DMA] * 5
        ),
    )(_sc_body)
    out_sc = sc_k(x2d, pe_table)

    out_tc = pl.pallas_call(
        _tc_body,
        grid=(S // S_BLK,),
        in_specs=[
            pl.BlockSpec((S_BLK, D), lambda s: (B_SC * (S // S_BLK) + s, 0)),
            pl.BlockSpec((S_BLK, D), lambda s: (s, 0)),
        ],
        out_specs=pl.BlockSpec((S_BLK, D), lambda s: (s, 0)),
        out_shape=jax.ShapeDtypeStruct((S, D), jnp.float32),
    )(x2d, pe_table)

    out2d = jnp.concatenate([out_sc, out_tc], axis=0)
    return out2d.reshape(B, S, D)


# speedup vs baseline: 1.6430x; 1.6430x over previous
import functools

import jax
import jax.numpy as jnp
from jax import lax
from jax.experimental import pallas as pl
from jax.experimental.pallas import tpu as pltpu
from jax.experimental.pallas import tpu_sc as plsc

B, S, D = 4, 8192, 1024
NC, NS = 2, 16
NW = NC * NS
S_PER_W = S // NW
CHUNK = 16
N_CHUNKS = S_PER_W // CHUNK
BPG = 2
LANES = 16


def _sc_body(x_hbm, pe_hbm, out_hbm,
             peb0, peb1,
             xb00, xb01, xb10, xb11,
             pe_sem0, pe_sem1, in_sem0, in_sem1, out_sem):
    peb = (peb0, peb1)
    xb = ((xb00, xb01), (xb10, xb11))
    pe_sem = (pe_sem0, pe_sem1)
    in_sem = (in_sem0, in_sem1)

    wid = lax.axis_index("s") * NC + lax.axis_index("c")
    base = wid * S_PER_W

    def pe_row(c):
        return pl.multiple_of(base + c * CHUNK, 8)

    def x_row(c, b):
        return pl.multiple_of(b * S + base + c * CHUNK, 8)

    def issue_pe(c, p):
        pltpu.async_copy(pe_hbm.at[pl.ds(pe_row(c), CHUNK)], peb[p], pe_sem[p])

    def issue_in(c, g, q):
        for i in range(BPG):
            pltpu.async_copy(x_hbm.at[pl.ds(x_row(c, BPG * g + i), CHUNK)],
                             xb[q][i], in_sem[q])

    def wait_pe(p):
        pltpu.make_async_copy(pe_hbm.at[pl.ds(0, CHUNK)], peb[p],
                              pe_sem[p]).wait()

    def wait_in(q):
        for i in range(BPG):
            pltpu.make_async_copy(x_hbm.at[pl.ds(0, CHUNK)], xb[q][i],
                                  in_sem[q]).wait()

    def drain_outs():
        for i in range(BPG):
            pltpu.make_async_copy(x_hbm.at[pl.ds(0, CHUNK)], xb[0][i],
                                  out_sem).wait()

    def step(c, g, q, p):
        if g == 0:
            wait_pe(p)

            @pl.when(c + 1 < N_CHUNKS)
            def _():
                issue_pe(c + 1, 1 - p)

        wait_in(q)

        @pl.when(2 * c + g > 0)
        def _():
            drain_outs()

        if g == 0:
            issue_in(c, 1, 1 - q)
        else:
            @pl.when(c + 1 < N_CHUNKS)
            def _():
                issue_in(c + 1, 0, 1 - q)

        bufs = xb[q]
        pbuf = peb[p]

        for r in range(CHUNK):
            @plsc.parallel_loop(0, D, LANES, unroll=4)
            def _(i, r=r):
                j = pl.multiple_of(i, 8)
                pe_slice = pbuf[r, pl.ds(j, LANES)]
                for b in range(BPG):
                    bufs[b][r, pl.ds(j, LANES)] = (
                        bufs[b][r, pl.ds(j, LANES)] + pe_slice)

        for i in range(BPG):
            pltpu.async_copy(bufs[i],
                             out_hbm.at[pl.ds(x_row(c, BPG * g + i), CHUNK)],
                             out_sem)

    issue_pe(0, 0)
    issue_in(0, 0, 0)

    def loop_body(t, carry):
        c0 = 2 * t
        step(c0, 0, 0, 0)
        step(c0, 1, 1, 0)
        step(c0 + 1, 0, 0, 1)
        step(c0 + 1, 1, 1, 1)
        return carry

    lax.fori_loop(0, N_CHUNKS // 2, loop_body, 0)
    drain_outs()


@jax.jit
def kernel(x, pe_table):
    mesh = plsc.VectorSubcoreMesh(core_axis_name="c", subcore_axis_name="s")
    k = functools.partial(
        pl.kernel,
        mesh=mesh,
        out_type=jax.ShapeDtypeStruct((B * S, D), jnp.float32),
        scratch_types=(
            [pltpu.VMEM((CHUNK, D), jnp.float32)] * 2
            + [pltpu.VMEM((CHUNK, D), jnp.float32)] * 4
            + [pltpu.SemaphoreType.DMA] * 5
        ),
    )(_sc_body)
    out2d = k(x.reshape(B * S, D), pe_table)
    return out2d.reshape(B, S, D)


# speedup vs baseline: 1.6966x; 1.0326x over previous
import functools

import jax
import jax.numpy as jnp
from jax import lax
from jax.experimental import pallas as pl
from jax.experimental.pallas import tpu as pltpu
from jax.experimental.pallas import tpu_sc as plsc

B, S, D = 4, 8192, 1024
NC, NS = 2, 16
NW = NC * NS
S_PER_W = S // NW
CHUNK = 8
N_CHUNKS = S_PER_W // CHUNK
LANES = 16


def _sc_body(x_hbm, pe_hbm, out_hbm,
             peb0, peb1,
             xb00, xb01, xb02, xb03,
             xb10, xb11, xb12, xb13,
             pe_sem0, pe_sem1, in_sem0, in_sem1, out_sem):
    peb = (peb0, peb1)
    xb = ((xb00, xb01, xb02, xb03), (xb10, xb11, xb12, xb13))
    pe_sem = (pe_sem0, pe_sem1)
    in_sem = (in_sem0, in_sem1)

    wid = lax.axis_index("s") * NC + lax.axis_index("c")
    base = wid * S_PER_W

    def pe_row(c):
        return pl.multiple_of(base + c * CHUNK, 8)

    def x_row(c, b):
        return pl.multiple_of(b * S + base + c * CHUNK, 8)

    def issue_pe(c, p):
        pltpu.async_copy(pe_hbm.at[pl.ds(pe_row(c), CHUNK)], peb[p], pe_sem[p])

    def issue_in(c, p):
        for b in range(B):
            pltpu.async_copy(x_hbm.at[pl.ds(x_row(c, b), CHUNK)], xb[p][b],
                             in_sem[p])

    def wait_pe(p):
        pltpu.make_async_copy(pe_hbm.at[pl.ds(0, CHUNK)], peb[p],
                              pe_sem[p]).wait()

    def wait_in(p):
        for b in range(B):
            pltpu.make_async_copy(x_hbm.at[pl.ds(0, CHUNK)], xb[p][b],
                                  in_sem[p]).wait()

    def drain_outs():
        for b in range(B):
            pltpu.make_async_copy(x_hbm.at[pl.ds(0, CHUNK)], xb[0][b],
                                  out_sem).wait()

    def chunk_step(c, p):
        wait_pe(p)

        @pl.when(c + 1 < N_CHUNKS)
        def _():
            issue_pe(c + 1, 1 - p)

        wait_in(p)

        @pl.when(c > 0)
        def _():
            drain_outs()

        @pl.when(c + 1 < N_CHUNKS)
        def _():
            issue_in(c + 1, 1 - p)

        bufs = xb[p]
        pbuf = peb[p]

        for r in range(CHUNK):
            @plsc.parallel_loop(0, D, LANES, unroll=4)
            def _(i, r=r):
                j = pl.multiple_of(i, 8)
                pe_slice = pbuf[r, pl.ds(j, LANES)]
                for b in range(B):
                    bufs[b][r, pl.ds(j, LANES)] = (
                        bufs[b][r, pl.ds(j, LANES)] + pe_slice)

        for b in range(B):
            pltpu.async_copy(bufs[b], out_hbm.at[pl.ds(x_row(c, b), CHUNK)],
                             out_sem)

    issue_pe(0, 0)
    issue_in(0, 0)

    def loop_body(t, carry):
        chunk_step(2 * t, 0)
        chunk_step(2 * t + 1, 1)
        return carry

    lax.fori_loop(0, N_CHUNKS // 2, loop_body, 0)
    drain_outs()


@jax.jit
def kernel(x, pe_table):
    mesh = plsc.VectorSubcoreMesh(core_axis_name="c", subcore_axis_name="s")
    k = functools.partial(
        pl.kernel,
        mesh=mesh,
        out_type=jax.ShapeDtypeStruct((B * S, D), jnp.float32),
        scratch_types=(
            [pltpu.VMEM((CHUNK, D), jnp.float32)] * 2
            + [pltpu.VMEM((CHUNK, D), jnp.float32)] * 8
            + [pltpu.SemaphoreType.DMA] * 5
        ),
    )(_sc_body)
    out2d = k(x.reshape(B * S, D), pe_table)
    return out2d.reshape(B, S, D)
